# Initial kernel scaffold; baseline (speedup 1.0000x reference)
import jax
import jax.numpy as jnp
from jax.experimental import pallas as pl


def kernel(x, edge_index, pairs, W1, b1, W2, b2, W3, b3, Wh1, bh1, Wh2, bh2):
    raise NotImplementedError("write your pallas kernel here")


# speedup vs baseline: 9.0371x; 9.0371x over previous
import jax
import jax.numpy as jnp
from jax import lax
from jax.experimental import pallas as pl
from jax.experimental.pallas import tpu as pltpu
from jax.experimental.pallas import tpu_sc as plsc

N_NODES = 10000
N_PAD = 10240
D_FEAT = 128
D1, D2, D3 = 64, 32, 16
N_EDGES = 320000
NW = 32
BLK = 128
EDGE_BLOCKS_PER_W = 80
E_PAD = NW * EDGE_BLOCKS_PER_W * BLK
N_PAIRS = 50000
PAIR_BLOCKS_PER_W = 13
P_PAD = NW * PAIR_BLOCKS_PER_W * BLK
ROWS_PER_TILE = N_PAD // 16


def _mesh():
    return plsc.VectorSubcoreMesh(core_axis_name="c", subcore_axis_name="s")


def _make_agg(d, with_deg):
    out_type = [jax.ShapeDtypeStruct((2, N_PAD, d), jnp.float32)]
    scratch = [
        pltpu.VMEM((EDGE_BLOCKS_PER_W, BLK), jnp.int32),
        pltpu.VMEM((EDGE_BLOCKS_PER_W, BLK), jnp.int32),
        pltpu.VMEM((BLK, d), jnp.float32),
        pltpu.VMEM_SHARED((N_PAD, d), jnp.float32),
        pltpu.SemaphoreType.DMA,
    ]
    if with_deg:
        out_type.append(jax.ShapeDtypeStruct((2, N_PAD), jnp.float32))
        scratch += [
            pltpu.VMEM((BLK,), jnp.float32),
            pltpu.VMEM_SHARED((N_PAD,), jnp.float32),
        ]

    def body(*refs):
        if with_deg:
            (t_hbm, srcm, dstm, zf, zd,
             part, degp, idx_s, idx_d, rows, acc, sem, ones, dega) = refs
        else:
            (t_hbm, srcm, dstm, zf,
             part, idx_s, idx_d, rows, acc, sem) = refs
        c = lax.axis_index("c")
        s = lax.axis_index("s")
        wid = s * 2 + c
        r0 = s * ROWS_PER_TILE
        pltpu.sync_copy(zf.at[pl.ds(r0, ROWS_PER_TILE)],
                        acc.at[pl.ds(r0, ROWS_PER_TILE)])
        if with_deg:
            pltpu.sync_copy(zd.at[pl.ds(r0, ROWS_PER_TILE)],
                            dega.at[pl.ds(r0, ROWS_PER_TILE)])
            for i in range(BLK // 16):
                ones[pl.ds(i * 16, 16)] = jnp.full((16,), 1.0, jnp.float32)
        pltpu.sync_copy(srcm.at[pl.ds(wid * EDGE_BLOCKS_PER_W,
                                      EDGE_BLOCKS_PER_W)], idx_s)
        pltpu.sync_copy(dstm.at[pl.ds(wid * EDGE_BLOCKS_PER_W,
                                      EDGE_BLOCKS_PER_W)], idx_d)
        plsc.subcore_barrier()

        def step(j, carry):
            pltpu.async_copy(t_hbm.at[idx_s.at[j]], rows, sem).wait()
            pltpu.sync_copy(rows, acc.at[idx_d.at[j]], add=True)
            if with_deg:
                pltpu.sync_copy(ones, dega.at[idx_d.at[j]], add=True)
            return carry

        lax.fori_loop(0, EDGE_BLOCKS_PER_W, step, 0)
        plsc.subcore_barrier()
        pltpu.sync_copy(acc.at[pl.ds(r0, ROWS_PER_TILE)],
                        part.at[c].at[pl.ds(r0, ROWS_PER_TILE)])
        if with_deg:
            pltpu.sync_copy(dega.at[pl.ds(r0, ROWS_PER_TILE)],
                            degp.at[c].at[pl.ds(r0, ROWS_PER_TILE)])

    def agg(*args):
        return pl.kernel(body, out_type=tuple(out_type), mesh=_mesh(),
                         compiler_params=pltpu.CompilerParams(
                             use_tc_tiling_on_sc=False),
                         scratch_types=tuple(scratch))(*args)
    return agg


def _pair_gather_body(h3, pam, pbm, ga, gb, idx_a, idx_b, rows, sem):
    c = lax.axis_index("c")
    s = lax.axis_index("s")
    wid = s * 2 + c
    pltpu.sync_copy(pam.at[pl.ds(wid * PAIR_BLOCKS_PER_W,
                                 PAIR_BLOCKS_PER_W)], idx_a)
    pltpu.sync_copy(pbm.at[pl.ds(wid * PAIR_BLOCKS_PER_W,
                                 PAIR_BLOCKS_PER_W)], idx_b)
    base = wid * PAIR_BLOCKS_PER_W * BLK

    def step_a(j, carry):
        pltpu.async_copy(h3.at[idx_a.at[j]], rows, sem).wait()
        pltpu.sync_copy(rows, ga.at[pl.ds(base + j * BLK, BLK)])
        return carry

    def step_b(j, carry):
        pltpu.async_copy(h3.at[idx_b.at[j]], rows, sem).wait()
        pltpu.sync_copy(rows, gb.at[pl.ds(base + j * BLK, BLK)])
        return carry

    lax.fori_loop(0, PAIR_BLOCKS_PER_W, step_a, 0)
    lax.fori_loop(0, PAIR_BLOCKS_PER_W, step_b, 0)


def _pair_gather(h3, pam, pbm):
    out_type = (jax.ShapeDtypeStruct((P_PAD, D3), jnp.float32),
                jax.ShapeDtypeStruct((P_PAD, D3), jnp.float32))
    scratch = (
        pltpu.VMEM((PAIR_BLOCKS_PER_W, BLK), jnp.int32),
        pltpu.VMEM((PAIR_BLOCKS_PER_W, BLK), jnp.int32),
        pltpu.VMEM((BLK, D3), jnp.float32),
        pltpu.SemaphoreType.DMA,
    )
    return pl.kernel(_pair_gather_body, out_type=out_type, mesh=_mesh(),
                     compiler_params=pltpu.CompilerParams(
                         use_tc_tiling_on_sc=False),
                     scratch_types=scratch)(h3, pam, pbm)


def _make_layer_body(act):
    def body(p_ref, degt_ref, w_ref, b_ref, o_ref):
        acc = p_ref[0] + p_ref[1]
        dsum = degt_ref[:, 0:1] + degt_ref[:, 1:2]
        aggn = acc / jnp.maximum(dsum, 1.0)
        out = jnp.dot(aggn, w_ref[...],
                      preferred_element_type=jnp.float32) + b_ref[...]
        o_ref[...] = jnp.maximum(out, 0.0) if act else out
    return body


def _head_body(ga_ref, gb_ref, w1_ref, b1_ref, w2_ref, b2_ref, o_ref):
    feat = jnp.abs(ga_ref[...] - gb_ref[...])
    z = jnp.maximum(
        jnp.dot(feat, w1_ref[...],
                preferred_element_type=jnp.float32) + b1_ref[...], 0.0)
    o_ref[...] = jnp.dot(z, w2_ref[...],
                         preferred_element_type=jnp.float32) + b2_ref[...]


def _tc_call(body, out_shape, *args):
    return pl.pallas_call(
        body, out_shape=jax.ShapeDtypeStruct(out_shape, jnp.float32))(*args)


@jax.jit
def kernel(x, edge_index, pairs, W1, b1, W2, b2, W3, b3, Wh1, bh1, Wh2, bh2):
    x_pad = jnp.pad(x, ((0, N_PAD - N_NODES), (0, 0)))

    n_epad = E_PAD - N_EDGES
    pad_e = (N_NODES + (jnp.arange(n_epad, dtype=jnp.int32) % 128))
    srcm = jnp.concatenate([edge_index[0].astype(jnp.int32), pad_e]
                           ).reshape(E_PAD // BLK, BLK)
    dstm = jnp.concatenate([edge_index[1].astype(jnp.int32), pad_e]
                           ).reshape(E_PAD // BLK, BLK)

    n_ppad = P_PAD - N_PAIRS
    pad_p = (N_NODES + (jnp.arange(n_ppad, dtype=jnp.int32) % 128))
    pam = jnp.concatenate([pairs[:, 0].astype(jnp.int32), pad_p]
                          ).reshape(P_PAD // BLK, BLK)
    pbm = jnp.concatenate([pairs[:, 1].astype(jnp.int32), pad_p]
                          ).reshape(P_PAD // BLK, BLK)

    zd = jnp.zeros((N_PAD,), jnp.float32)
    z128 = jnp.zeros((N_PAD, D_FEAT), jnp.float32)
    z64 = jnp.zeros((N_PAD, D1), jnp.float32)
    z32 = jnp.zeros((N_PAD, D2), jnp.float32)

    agg128 = _make_agg(D_FEAT, with_deg=True)
    agg64 = _make_agg(D1, with_deg=False)
    agg32 = _make_agg(D2, with_deg=False)

    p1, degp = agg128(x_pad, srcm, dstm, z128, zd)
    degt = degp.T

    h1 = _tc_call(_make_layer_body(True), (N_PAD, D1), p1, degt, W1,
                  b1.reshape(1, D1))
    p2 = agg64(h1, srcm, dstm, z64)[0]

    h2 = _tc_call(_make_layer_body(True), (N_PAD, D2), p2, degt, W2,
                  b2.reshape(1, D2))
    p3 = agg32(h2, srcm, dstm, z32)[0]

    h3 = _tc_call(_make_layer_body(False), (N_PAD, D3), p3, degt, W3,
                  b3.reshape(1, D3))

    ga, gb = _pair_gather(h3, pam, pbm)

    hb = 4096
    logits = pl.pallas_call(
        _head_body,
        grid=(P_PAD // hb,),
        in_specs=[
            pl.BlockSpec((hb, D3), lambda i: (i, 0)),
            pl.BlockSpec((hb, D3), lambda i: (i, 0)),
            pl.BlockSpec((D3, 32), lambda i: (0, 0)),
            pl.BlockSpec((1, 32), lambda i: (0, 0)),
            pl.BlockSpec((32, 1), lambda i: (0, 0)),
            pl.BlockSpec((1, 1), lambda i: (0, 0)),
        ],
        out_specs=pl.BlockSpec((hb, 1), lambda i: (i, 0)),
        out_shape=jax.ShapeDtypeStruct((P_PAD, 1), jnp.float32),
    )(ga, gb, Wh1, bh1.reshape(1, -1), Wh2, bh2.reshape(1, 1))
    return logits.reshape(-1)[:N_PAIRS]


# speedup vs baseline: 9.5801x; 1.0601x over previous
import jax
import jax.numpy as jnp
from jax import lax
from jax.experimental import pallas as pl
from jax.experimental.pallas import tpu as pltpu
from jax.experimental.pallas import tpu_sc as plsc

N_NODES = 10000
N_PAD = 10240
D_FEAT = 128
D1, D2, D3 = 64, 32, 16
N_EDGES = 320000
NW = 32
BLK = 128
EB = 80
E_PAD = NW * EB * BLK
N_PAIRS = 50000
PB = 26
P_PAD = NW * (PB // 2) * BLK
ROWS_PER_TILE = N_PAD // 16


def _mesh():
    return plsc.VectorSubcoreMesh(core_axis_name="c", subcore_axis_name="s")


CHUNK = 8
NCHUNK = EB // CHUNK


def _make_agg(d, with_deg):
    out_type = [jax.ShapeDtypeStruct((2, N_PAD, d), jnp.float32)]
    scratch = [
        pltpu.VMEM((2 * CHUNK, BLK), jnp.int32),
        pltpu.VMEM((2 * CHUNK, BLK), jnp.int32),
        pltpu.VMEM((2 * BLK, d), jnp.float32),
        pltpu.VMEM_SHARED((N_PAD, d), jnp.float32),
        pltpu.SemaphoreType.DMA,
        pltpu.SemaphoreType.DMA,
    ]
    if with_deg:
        out_type.append(jax.ShapeDtypeStruct((2, N_PAD), jnp.float32))
        scratch += [
            pltpu.VMEM((BLK,), jnp.float32),
            pltpu.VMEM_SHARED((N_PAD,), jnp.float32),
        ]

    def body(*refs):
        if with_deg:
            (t_hbm, srcm, dstm, zf, zd, part, degp,
             idx_s, idx_d, rows, acc, gsem0, gsem1, ones, dega) = refs
        else:
            (t_hbm, srcm, dstm, zf, part,
             idx_s, idx_d, rows, acc, gsem0, gsem1) = refs
        gsem = (gsem0, gsem1)
        c = lax.axis_index("c")
        s = lax.axis_index("s")
        wid = s * 2 + c
        r0 = s * ROWS_PER_TILE
        for k in range(ROWS_PER_TILE // BLK):
            pltpu.sync_copy(zf, acc.at[pl.ds(r0 + k * BLK, BLK)])
            if with_deg:
                pltpu.sync_copy(zd, dega.at[pl.ds(r0 + k * BLK, BLK)])
        if with_deg:
            for i in range(BLK // 16):
                ones[pl.ds(i * 16, 16)] = jnp.full((16,), 1.0, jnp.float32)
        plsc.subcore_barrier()

        def buf(b):
            return rows.at[pl.ds(b * BLK, BLK)]

        def stage(ch, half):
            pltpu.sync_copy(srcm.at[pl.ds(wid * EB + ch * CHUNK, CHUNK)],
                            idx_s.at[pl.ds(half * CHUNK, CHUNK)])
            pltpu.sync_copy(dstm.at[pl.ds(wid * EB + ch * CHUNK, CHUNK)],
                            idx_d.at[pl.ds(half * CHUNK, CHUNK)])

        def gather(row, b):
            pltpu.async_copy(t_hbm.at[idx_s.at[row]], buf(b), gsem[b])

        def gather_wait(row, b):
            pltpu.make_async_copy(t_hbm.at[idx_s.at[row]], buf(b),
                                  gsem[b]).wait()

        def scatter(row, b):
            pltpu.sync_copy(buf(b), acc.at[idx_d.at[row]], add=True)
            if with_deg:
                pltpu.sync_copy(ones, dega.at[idx_d.at[row]], add=True)

        stage(0, 0)
        gather(0, 0)

        def chunk_step(i, carry):
            half = lax.rem(i, 2)
            nxt = lax.rem(i + 1, 2)
            stage(lax.rem(i + 1, NCHUNK), nxt)
            for j in range(CHUNK):
                b = j % 2
                row = half * CHUNK + j
                nrow = row + 1 if j + 1 < CHUNK else nxt * CHUNK
                gather_wait(row, b)
                gather(nrow, 1 - b)
                scatter(row, b)
            return carry

        lax.fori_loop(0, NCHUNK, chunk_step, 0)
        gather_wait(0, 0)
        plsc.subcore_barrier()
        pltpu.sync_copy(acc.at[pl.ds(r0, ROWS_PER_TILE)],
                        part.at[c].at[pl.ds(r0, ROWS_PER_TILE)])
        if with_deg:
            pltpu.sync_copy(dega.at[pl.ds(r0, ROWS_PER_TILE)],
                            degp.at[c].at[pl.ds(r0, ROWS_PER_TILE)])

    def agg(*args):
        return pl.kernel(body, out_type=tuple(out_type), mesh=_mesh(),
                         compiler_params=pltpu.CompilerParams(
                             use_tc_tiling_on_sc=False),
                         scratch_types=tuple(scratch))(*args)
    return agg


def _pair_gather_body(h3, pm, gout, idx, rows, sem0, sem1):
    c = lax.axis_index("c")
    s = lax.axis_index("s")
    wid = s * 2 + c
    pltpu.sync_copy(pm.at[pl.ds(wid * PB, PB)], idx)
    base = wid * PB * BLK
    sems = (sem0, sem1)

    def gather(jj, b):
        pltpu.async_copy(h3.at[idx.at[jj]], rows.at[pl.ds(b * BLK, BLK)],
                         sems[b])

    def gather_wait(jj, b):
        pltpu.make_async_copy(h3.at[idx.at[jj]], rows.at[pl.ds(b * BLK, BLK)],
                              sems[b]).wait()

    gather(0, 0)
    gather(1, 1)

    def emit(jj, b):
        gather_wait(jj, b)
        pltpu.sync_copy(rows.at[pl.ds(b * BLK, BLK)],
                        gout.at[pl.ds(base + jj * BLK, BLK)])

    def step(i, carry):
        for b in range(2):
            jj = i * 2 + b
            emit(jj, b)
            gather(jj + 2, b)
        return carry

    lax.fori_loop(0, PB // 2 - 1, step, 0)
    emit(PB - 2, 0)
    emit(PB - 1, 1)


def _pair_gather(h3, pm):
    out_type = jax.ShapeDtypeStruct((NW * PB * BLK, D3), jnp.float32)
    scratch = (
        pltpu.VMEM((PB, BLK), jnp.int32),
        pltpu.VMEM((2 * BLK, D3), jnp.float32),
        pltpu.SemaphoreType.DMA,
        pltpu.SemaphoreType.DMA,
    )
    return pl.kernel(_pair_gather_body, out_type=out_type, mesh=_mesh(),
                     compiler_params=pltpu.CompilerParams(
                         use_tc_tiling_on_sc=False),
                     scratch_types=scratch)(h3, pm)


def _make_layer_body(act):
    def body(p_ref, degt_ref, w_ref, b_ref, o_ref):
        acc = p_ref[0] + p_ref[1]
        dsum = degt_ref[:, 0:1] + degt_ref[:, 1:2]
        aggn = acc / jnp.maximum(dsum, 1.0)
        out = jnp.dot(aggn, w_ref[...],
                      preferred_element_type=jnp.float32) + b_ref[...]
        o_ref[...] = jnp.maximum(out, 0.0) if act else out
    return body


def _head_body(ga_ref, gb_ref, w1_ref, b1_ref, w2_ref, b2_ref, o_ref):
    feat = jnp.abs(ga_ref[...] - gb_ref[...])
    z = jnp.maximum(
        jnp.dot(feat, w1_ref[...],
                preferred_element_type=jnp.float32) + b1_ref[...], 0.0)
    o_ref[...] = jnp.dot(z, w2_ref[...],
                         preferred_element_type=jnp.float32) + b2_ref[...]


def _tc_call(body, out_shape, *args):
    return pl.pallas_call(
        body, out_shape=jax.ShapeDtypeStruct(out_shape, jnp.float32))(*args)


@jax.jit
def kernel(x, edge_index, pairs, W1, b1, W2, b2, W3, b3, Wh1, bh1, Wh2, bh2):
    x_pad = jnp.pad(x, ((0, N_PAD - N_NODES), (0, 0)))

    n_epad = E_PAD - N_EDGES
    pad_e = (N_NODES + (jnp.arange(n_epad, dtype=jnp.int32) % 128))
    srcm = jnp.concatenate([edge_index[0].astype(jnp.int32), pad_e]
                           ).reshape(E_PAD // BLK, BLK)
    dstm = jnp.concatenate([edge_index[1].astype(jnp.int32), pad_e]
                           ).reshape(E_PAD // BLK, BLK)

    n_ppad = P_PAD - N_PAIRS
    pad_p = (N_NODES + (jnp.arange(n_ppad, dtype=jnp.int32) % 128))
    pcat = jnp.concatenate([pairs[:, 0].astype(jnp.int32), pad_p,
                            pairs[:, 1].astype(jnp.int32), pad_p])
    pm = pcat.reshape(2 * P_PAD // BLK, BLK)

    zd = jnp.zeros((BLK,), jnp.float32)
    z128 = jnp.zeros((BLK, D_FEAT), jnp.float32)
    z64 = jnp.zeros((BLK, D1), jnp.float32)
    z32 = jnp.zeros((BLK, D2), jnp.float32)

    agg128 = _make_agg(D_FEAT, with_deg=True)
    agg64 = _make_agg(D1, with_deg=False)
    agg32 = _make_agg(D2, with_deg=False)

    p1, degp = agg128(x_pad, srcm, dstm, z128, zd)
    degt = degp.T

    h1 = _tc_call(_make_layer_body(True), (N_PAD, D1), p1, degt, W1,
                  b1.reshape(1, D1))
    p2 = agg64(h1, srcm, dstm, z64)[0]

    h2 = _tc_call(_make_layer_body(True), (N_PAD, D2), p2, degt, W2,
                  b2.reshape(1, D2))
    p3 = agg32(h2, srcm, dstm, z32)[0]

    h3 = _tc_call(_make_layer_body(False), (N_PAD, D3), p3, degt, W3,
                  b3.reshape(1, D3))

    gout = _pair_gather(h3, pm)

    ga = gout[:P_PAD]
    gb = gout[P_PAD:]

    hb = 4096
    logits = pl.pallas_call(
        _head_body,
        grid=(P_PAD // hb,),
        in_specs=[
            pl.BlockSpec((hb, D3), lambda i: (i, 0)),
            pl.BlockSpec((hb, D3), lambda i: (i, 0)),
            pl.BlockSpec((D3, 32), lambda i: (0, 0)),
            pl.BlockSpec((1, 32), lambda i: (0, 0)),
            pl.BlockSpec((32, 1), lambda i: (0, 0)),
            pl.BlockSpec((1, 1), lambda i: (0, 0)),
        ],
        out_specs=pl.BlockSpec((hb, 1), lambda i: (i, 0)),
        out_shape=jax.ShapeDtypeStruct((P_PAD, 1), jnp.float32),
    )(ga, gb, Wh1, bh1.reshape(1, -1), Wh2, bh2.reshape(1, 1))
    return logits.reshape(-1)[:N_PAIRS]


# speedup vs baseline: 10.9222x; 1.1401x over previous
import jax
import jax.numpy as jnp
from jax import lax
from jax.experimental import pallas as pl
from jax.experimental.pallas import tpu as pltpu
from jax.experimental.pallas import tpu_sc as plsc

N_NODES = 10000
N_PAD = 10240
D_FEAT = 128
D1, D2, D3 = 64, 32, 16
N_EDGES = 320000
NW = 32
E_W = 10240
E_PAD = NW * E_W
N_PAIRS = 50000
P_W = 3328
P_PAD = NW * P_W // 2
PW_BLK = 256
ROWS_PER_TILE = N_PAD // 16
ZROWS = 128
CHB = 4


def _mesh():
    return plsc.VectorSubcoreMesh(core_axis_name="c", subcore_axis_name="s")


def _make_agg(d, w, with_deg):
    blocks = E_W // w
    nch = blocks // CHB
    cw = CHB * w
    sb = w // 128
    out_type = [jax.ShapeDtypeStruct((2, N_PAD, d), jnp.float32)]
    scratch = [
        pltpu.VMEM((2 * cw,), jnp.int32),
        pltpu.VMEM((2 * CHB * sb, 128), jnp.int32),
        pltpu.VMEM((2 * w, d), jnp.float32),
        pltpu.VMEM_SHARED((N_PAD, d), jnp.float32),
        pltpu.SemaphoreType.DMA,
        pltpu.SemaphoreType.DMA,
    ]
    if with_deg:
        out_type.append(jax.ShapeDtypeStruct((2, N_PAD), jnp.float32))
        scratch += [
            pltpu.VMEM((128,), jnp.float32),
            pltpu.VMEM_SHARED((N_PAD,), jnp.float32),
        ]

    def body(*refs):
        if with_deg:
            (t_hbm, srcm, dstm, zf, zd, part, degp,
             idx_s, idx_d, rows, acc, gsem0, gsem1, ones, dega) = refs
        else:
            (t_hbm, srcm, dstm, zf, part,
             idx_s, idx_d, rows, acc, gsem0, gsem1) = refs
        gsem = (gsem0, gsem1)
        c = lax.axis_index("c")
        s = lax.axis_index("s")
        wid = s * 2 + c
        r0 = s * ROWS_PER_TILE
        for k in range(ROWS_PER_TILE // ZROWS):
            pltpu.sync_copy(zf, acc.at[pl.ds(r0 + k * ZROWS, ZROWS)])
            if with_deg:
                pltpu.sync_copy(zd, dega.at[pl.ds(r0 + k * ZROWS, ZROWS)])
        if with_deg:
            for i in range(128 // 16):
                ones[pl.ds(i * 16, 16)] = jnp.full((16,), 1.0, jnp.float32)
        plsc.subcore_barrier()

        def buf(b):
            return rows.at[pl.ds(b * w, w)]

        def stage(ch, half):
            pltpu.sync_copy(srcm.at[pl.ds(wid * E_W + ch * cw, cw)],
                            idx_s.at[pl.ds(half * cw, cw)])
            pltpu.sync_copy(
                dstm.at[pl.ds((wid * E_W + ch * cw) // 128, CHB * sb)],
                idx_d.at[pl.ds(half * CHB * sb, CHB * sb)])

        def gather(pos, b):
            pltpu.async_copy(t_hbm.at[idx_s.at[pl.ds(pos, w)]], buf(b),
                             gsem[b])

        def gather_wait(pos, b):
            pltpu.make_async_copy(t_hbm.at[idx_s.at[pl.ds(pos, w)]], buf(b),
                                  gsem[b]).wait()

        def scatter(half, j, b):
            for q in range(sb):
                drow = half * CHB * sb + j * sb + q
                pltpu.sync_copy(rows.at[pl.ds(b * w + q * 128, 128)],
                                acc.at[idx_d.at[drow]], add=True)
                if with_deg:
                    pltpu.sync_copy(ones, dega.at[idx_d.at[drow]], add=True)

        stage(0, 0)
        gather(0, 0)

        def chunk_step(i, carry):
            half = lax.rem(i, 2)
            nxt = lax.rem(i + 1, 2)
            stage(lax.rem(i + 1, nch), nxt)
            for j in range(CHB):
                b = j % 2
                pos = half * cw + j * w
                npos = pos + w if j + 1 < CHB else nxt * cw
                gather_wait(pos, b)
                gather(npos, 1 - b)
                scatter(half, j, b)
            return carry

        lax.fori_loop(0, nch, chunk_step, 0)
        gather_wait(0, 0)
        plsc.subcore_barrier()
        pltpu.sync_copy(acc.at[pl.ds(r0, ROWS_PER_TILE)],
                        part.at[c].at[pl.ds(r0, ROWS_PER_TILE)])
        if with_deg:
            pltpu.sync_copy(dega.at[pl.ds(r0, ROWS_PER_TILE)],
                            degp.at[c].at[pl.ds(r0, ROWS_PER_TILE)])

    def agg(*args):
        return pl.kernel(body, out_type=tuple(out_type), mesh=_mesh(),
                         compiler_params=pltpu.CompilerParams(
                             use_tc_tiling_on_sc=False),
                         scratch_types=tuple(scratch))(*args)
    return agg


def _pair_gather_body(h3, pm, gout, idx, rows, sem0, sem1):
    c = lax.axis_index("c")
    s = lax.axis_index("s")
    wid = s * 2 + c
    pltpu.sync_copy(pm.at[pl.ds(wid * P_W, P_W)], idx)
    base = wid * P_W
    sems = (sem0, sem1)
    nb = P_W // PW_BLK

    def ib(j):
        return idx.at[pl.ds(j * PW_BLK, PW_BLK)]

    def buf(b):
        return rows.at[pl.ds(b * PW_BLK, PW_BLK)]

    def gather(j, b):
        pltpu.async_copy(h3.at[ib(j)], buf(b), sems[b])

    def emit(j, b):
        pltpu.make_async_copy(h3.at[ib(j)], buf(b), sems[b]).wait()
        pltpu.sync_copy(buf(b), gout.at[pl.ds(base + j * PW_BLK, PW_BLK)])

    gather(0, 0)
    gather(1, 1)

    def step(i, carry):
        j = i * 2
        emit(j, 0)
        gather(j + 2, 0)
        emit(j + 1, 1)
        gather(j + 3, 1)
        return carry

    lax.fori_loop(0, (nb - 3) // 2, step, 0)
    emit(nb - 3, 0)
    gather(nb - 1, 0)
    emit(nb - 2, 1)
    emit(nb - 1, 0)


def _pair_gather(h3, pm):
    out_type = jax.ShapeDtypeStruct((NW * P_W, D3), jnp.float32)
    scratch = (
        pltpu.VMEM((P_W,), jnp.int32),
        pltpu.VMEM((2 * PW_BLK, D3), jnp.float32),
        pltpu.SemaphoreType.DMA,
        pltpu.SemaphoreType.DMA,
    )
    return pl.kernel(_pair_gather_body, out_type=out_type, mesh=_mesh(),
                     compiler_params=pltpu.CompilerParams(
                         use_tc_tiling_on_sc=False),
                     scratch_types=scratch)(h3, pm)


def _make_layer_body(act):
    def body(p_ref, degt_ref, w_ref, b_ref, o_ref):
        acc = p_ref[0] + p_ref[1]
        dsum = degt_ref[:, 0:1] + degt_ref[:, 1:2]
        aggn = acc / jnp.maximum(dsum, 1.0)
        out = jnp.dot(aggn, w_ref[...],
                      preferred_element_type=jnp.float32) + b_ref[...]
        o_ref[...] = jnp.maximum(out, 0.0) if act else out
    return body


def _head_body(ga_ref, gb_ref, w1_ref, b1_ref, w2_ref, b2_ref, o_ref):
    feat = jnp.abs(ga_ref[...] - gb_ref[...])
    z = jnp.maximum(
        jnp.dot(feat, w1_ref[...],
                preferred_element_type=jnp.float32) + b1_ref[...], 0.0)
    o_ref[...] = jnp.dot(z, w2_ref[...],
                         preferred_element_type=jnp.float32) + b2_ref[...]


def _tc_call(body, out_shape, *args):
    return pl.pallas_call(
        body, out_shape=jax.ShapeDtypeStruct(out_shape, jnp.float32))(*args)


@jax.jit
def kernel(x, edge_index, pairs, W1, b1, W2, b2, W3, b3, Wh1, bh1, Wh2, bh2):
    x_pad = jnp.pad(x, ((0, N_PAD - N_NODES), (0, 0)))

    n_epad = E_PAD - N_EDGES
    pad_e = (N_NODES + (jnp.arange(n_epad, dtype=jnp.int32) % 128))
    srcm = jnp.concatenate([edge_index[0].astype(jnp.int32), pad_e])
    dstm = jnp.concatenate([edge_index[1].astype(jnp.int32), pad_e]
                           ).reshape(E_PAD // 128, 128)

    n_ppad = P_PAD - N_PAIRS
    pad_p = (N_NODES + (jnp.arange(n_ppad, dtype=jnp.int32) % 128))
    pcat = jnp.concatenate([pairs[:, 0].astype(jnp.int32), pad_p,
                            pairs[:, 1].astype(jnp.int32), pad_p])

    zd = jnp.zeros((ZROWS,), jnp.float32)
    z128 = jnp.zeros((ZROWS, D_FEAT), jnp.float32)
    z64 = jnp.zeros((ZROWS, D1), jnp.float32)
    z32 = jnp.zeros((ZROWS, D2), jnp.float32)

    agg128 = _make_agg(D_FEAT, 128, with_deg=True)
    agg64 = _make_agg(D1, 512, with_deg=False)
    agg32 = _make_agg(D2, 512, with_deg=False)

    p1, degp = agg128(x_pad, srcm, dstm, z128, zd)
    degt = degp.T

    h1 = _tc_call(_make_layer_body(True), (N_PAD, D1), p1, degt, W1,
                  b1.reshape(1, D1))
    p2 = agg64(h1, srcm, dstm, z64)[0]

    h2 = _tc_call(_make_layer_body(True), (N_PAD, D2), p2, degt, W2,
                  b2.reshape(1, D2))
    p3 = agg32(h2, srcm, dstm, z32)[0]

    h3 = _tc_call(_make_layer_body(False), (N_PAD, D3), p3, degt, W3,
                  b3.reshape(1, D3))

    gout = _pair_gather(h3, pcat)

    ga = gout[:P_PAD]
    gb = gout[P_PAD:]

    hb = 4096
    logits = pl.pallas_call(
        _head_body,
        grid=(P_PAD // hb,),
        in_specs=[
            pl.BlockSpec((hb, D3), lambda i: (i, 0)),
            pl.BlockSpec((hb, D3), lambda i: (i, 0)),
            pl.BlockSpec((D3, 32), lambda i: (0, 0)),
            pl.BlockSpec((1, 32), lambda i: (0, 0)),
            pl.BlockSpec((32, 1), lambda i: (0, 0)),
            pl.BlockSpec((1, 1), lambda i: (0, 0)),
        ],
        out_specs=pl.BlockSpec((hb, 1), lambda i: (i, 0)),
        out_shape=jax.ShapeDtypeStruct((P_PAD, 1), jnp.float32),
    )(ga, gb, Wh1, bh1.reshape(1, -1), Wh2, bh2.reshape(1, 1))
    return logits.reshape(-1)[:N_PAIRS]


# speedup vs baseline: 11.3410x; 1.0383x over previous
import jax
import jax.numpy as jnp
from jax import lax
from jax.experimental import pallas as pl
from jax.experimental.pallas import tpu as pltpu
from jax.experimental.pallas import tpu_sc as plsc

N_NODES = 10000
N_PAD = 10240
D_FEAT = 128
D1, D2, D3 = 64, 32, 16
N_EDGES = 320000
NW = 32
E_W = 10240
E_PAD = NW * E_W
N_PAIRS = 50000
P_W = 3328
P_PAD = NW * P_W // 2
PW_BLK = 256
ROWS_PER_TILE = N_PAD // 16
ZROWS = 640
CHB = 4


def _mesh():
    return plsc.VectorSubcoreMesh(core_axis_name="c", subcore_axis_name="s")


def _make_agg(d, w, with_deg):
    blocks = E_W // w
    nch = blocks // CHB
    cw = CHB * w
    sb = w // 128
    out_type = [jax.ShapeDtypeStruct((2, N_PAD, d), jnp.float32)]
    scratch = [
        pltpu.VMEM((2 * cw,), jnp.int32),
        pltpu.VMEM((2 * CHB * sb, 128), jnp.int32),
        pltpu.VMEM((2 * w, d), jnp.float32),
        pltpu.VMEM_SHARED((N_PAD, d), jnp.float32),
        pltpu.SemaphoreType.DMA,
        pltpu.SemaphoreType.DMA,
    ]
    if with_deg:
        out_type.append(jax.ShapeDtypeStruct((2, N_PAD), jnp.float32))
        scratch += [
            pltpu.VMEM((128,), jnp.float32),
            pltpu.VMEM_SHARED((N_PAD,), jnp.float32),
        ]

    def body(*refs):
        if with_deg:
            (t_hbm, srcm, dstm, zf, zd, part, degp,
             idx_s, idx_d, rows, acc, gsem0, gsem1, ones, dega) = refs
        else:
            (t_hbm, srcm, dstm, zf, part,
             idx_s, idx_d, rows, acc, gsem0, gsem1) = refs
        gsem = (gsem0, gsem1)
        c = lax.axis_index("c")
        s = lax.axis_index("s")
        wid = s * 2 + c
        r0 = s * ROWS_PER_TILE
        for k in range(ROWS_PER_TILE // ZROWS):
            pltpu.sync_copy(zf, acc.at[pl.ds(r0 + k * ZROWS, ZROWS)])
            if with_deg:
                pltpu.sync_copy(zd, dega.at[pl.ds(r0 + k * ZROWS, ZROWS)])
        if with_deg:
            for i in range(128 // 16):
                ones[pl.ds(i * 16, 16)] = jnp.full((16,), 1.0, jnp.float32)
        plsc.subcore_barrier()

        def buf(b):
            return rows.at[pl.ds(b * w, w)]

        def stage(ch, half):
            pltpu.sync_copy(srcm.at[pl.ds(wid * E_W + ch * cw, cw)],
                            idx_s.at[pl.ds(half * cw, cw)])
            pltpu.sync_copy(
                dstm.at[pl.ds((wid * E_W + ch * cw) // 128, CHB * sb)],
                idx_d.at[pl.ds(half * CHB * sb, CHB * sb)])

        def gather(pos, b):
            pltpu.async_copy(t_hbm.at[idx_s.at[pl.ds(pos, w)]], buf(b),
                             gsem[b])

        def gather_wait(pos, b):
            pltpu.make_async_copy(t_hbm.at[idx_s.at[pl.ds(pos, w)]], buf(b),
                                  gsem[b]).wait()

        def scatter(half, j, b):
            for q in range(sb):
                drow = half * CHB * sb + j * sb + q
                pltpu.sync_copy(rows.at[pl.ds(b * w + q * 128, 128)],
                                acc.at[idx_d.at[drow]], add=True)
                if with_deg:
                    pltpu.sync_copy(ones, dega.at[idx_d.at[drow]], add=True)

        stage(0, 0)
        gather(0, 0)

        def chunk_step(i, carry):
            half = lax.rem(i, 2)
            nxt = lax.rem(i + 1, 2)
            stage(lax.rem(i + 1, nch), nxt)
            for j in range(CHB):
                b = j % 2
                pos = half * cw + j * w
                npos = pos + w if j + 1 < CHB else nxt * cw
                gather_wait(pos, b)
                gather(npos, 1 - b)
                scatter(half, j, b)
            return carry

        lax.fori_loop(0, nch, chunk_step, 0)
        gather_wait(0, 0)
        plsc.subcore_barrier()
        pltpu.sync_copy(acc.at[pl.ds(r0, ROWS_PER_TILE)],
                        part.at[c].at[pl.ds(r0, ROWS_PER_TILE)])
        if with_deg:
            pltpu.sync_copy(dega.at[pl.ds(r0, ROWS_PER_TILE)],
                            degp.at[c].at[pl.ds(r0, ROWS_PER_TILE)])

    def agg(*args):
        return pl.kernel(body, out_type=tuple(out_type), mesh=_mesh(),
                         compiler_params=pltpu.CompilerParams(
                             use_tc_tiling_on_sc=False),
                         scratch_types=tuple(scratch))(*args)
    return agg


def _pair_gather_body(h3, pm, gout, idx, rows, sem0, sem1):
    c = lax.axis_index("c")
    s = lax.axis_index("s")
    wid = s * 2 + c
    pltpu.sync_copy(pm.at[pl.ds(wid * P_W, P_W)], idx)
    base = wid * P_W
    sems = (sem0, sem1)
    nb = P_W // PW_BLK

    def ib(j):
        return idx.at[pl.ds(j * PW_BLK, PW_BLK)]

    def buf(b):
        return rows.at[pl.ds(b * PW_BLK, PW_BLK)]

    def gather(j, b):
        pltpu.async_copy(h3.at[ib(j)], buf(b), sems[b])

    def emit(j, b):
        pltpu.make_async_copy(h3.at[ib(j)], buf(b), sems[b]).wait()
        pltpu.sync_copy(buf(b), gout.at[pl.ds(base + j * PW_BLK, PW_BLK)])

    gather(0, 0)
    gather(1, 1)

    def step(i, carry):
        j = i * 2
        emit(j, 0)
        gather(j + 2, 0)
        emit(j + 1, 1)
        gather(j + 3, 1)
        return carry

    lax.fori_loop(0, (nb - 3) // 2, step, 0)
    emit(nb - 3, 0)
    gather(nb - 1, 0)
    emit(nb - 2, 1)
    emit(nb - 1, 0)


def _pair_gather(h3, pm):
    out_type = jax.ShapeDtypeStruct((NW * P_W, D3), jnp.float32)
    scratch = (
        pltpu.VMEM((P_W,), jnp.int32),
        pltpu.VMEM((2 * PW_BLK, D3), jnp.float32),
        pltpu.SemaphoreType.DMA,
        pltpu.SemaphoreType.DMA,
    )
    return pl.kernel(_pair_gather_body, out_type=out_type, mesh=_mesh(),
                     compiler_params=pltpu.CompilerParams(
                         use_tc_tiling_on_sc=False),
                     scratch_types=scratch)(h3, pm)


def _make_layer_body(act):
    def body(p_ref, degt_ref, w_ref, b_ref, o_ref):
        acc = p_ref[0] + p_ref[1]
        dsum = degt_ref[:, 0:1] + degt_ref[:, 1:2]
        aggn = acc / jnp.maximum(dsum, 1.0)
        out = jnp.dot(aggn, w_ref[...],
                      preferred_element_type=jnp.float32) + b_ref[...]
        o_ref[...] = jnp.maximum(out, 0.0) if act else out
    return body


def _head_body(ga_ref, gb_ref, w1_ref, b1_ref, w2_ref, b2_ref, o_ref):
    feat = jnp.abs(ga_ref[...] - gb_ref[...])
    z = jnp.maximum(
        jnp.dot(feat, w1_ref[...],
                preferred_element_type=jnp.float32) + b1_ref[...], 0.0)
    o_ref[...] = jnp.dot(z, w2_ref[...],
                         preferred_element_type=jnp.float32) + b2_ref[...]


def _tc_call(body, out_shape, *args):
    return pl.pallas_call(
        body, out_shape=jax.ShapeDtypeStruct(out_shape, jnp.float32))(*args)


@jax.jit
def kernel(x, edge_index, pairs, W1, b1, W2, b2, W3, b3, Wh1, bh1, Wh2, bh2):
    x_pad = jnp.pad(x, ((0, N_PAD - N_NODES), (0, 0)))

    n_epad = E_PAD - N_EDGES
    pad_e = (N_NODES + (jnp.arange(n_epad, dtype=jnp.int32) % 128))
    srcm = jnp.concatenate([edge_index[0].astype(jnp.int32), pad_e])
    dstm = jnp.concatenate([edge_index[1].astype(jnp.int32), pad_e]
                           ).reshape(E_PAD // 128, 128)

    n_ppad = P_PAD - N_PAIRS
    pad_p = (N_NODES + (jnp.arange(n_ppad, dtype=jnp.int32) % 128))
    pcat = jnp.concatenate([pairs[:, 0].astype(jnp.int32), pad_p,
                            pairs[:, 1].astype(jnp.int32), pad_p])

    zd = jnp.zeros((ZROWS,), jnp.float32)
    z128 = jnp.zeros((ZROWS, D_FEAT), jnp.float32)
    z64 = jnp.zeros((ZROWS, D1), jnp.float32)
    z32 = jnp.zeros((ZROWS, D2), jnp.float32)

    agg128 = _make_agg(D_FEAT, 128, with_deg=True)
    agg64 = _make_agg(D1, 512, with_deg=False)
    agg32 = _make_agg(D2, 512, with_deg=False)

    p1, degp = agg128(x_pad, srcm, dstm, z128, zd)
    degt = degp.T

    h1 = _tc_call(_make_layer_body(True), (N_PAD, D1), p1, degt, W1,
                  b1.reshape(1, D1))
    p2 = agg64(h1, srcm, dstm, z64)[0]

    h2 = _tc_call(_make_layer_body(True), (N_PAD, D2), p2, degt, W2,
                  b2.reshape(1, D2))
    p3 = agg32(h2, srcm, dstm, z32)[0]

    h3 = _tc_call(_make_layer_body(False), (N_PAD, D3), p3, degt, W3,
                  b3.reshape(1, D3))

    gout = _pair_gather(h3, pcat)

    ga = gout[:P_PAD]
    gb = gout[P_PAD:]

    hb = 4096
    logits = pl.pallas_call(
        _head_body,
        grid=(P_PAD // hb,),
        in_specs=[
            pl.BlockSpec((hb, D3), lambda i: (i, 0)),
            pl.BlockSpec((hb, D3), lambda i: (i, 0)),
            pl.BlockSpec((D3, 32), lambda i: (0, 0)),
            pl.BlockSpec((1, 32), lambda i: (0, 0)),
            pl.BlockSpec((32, 1), lambda i: (0, 0)),
            pl.BlockSpec((1, 1), lambda i: (0, 0)),
        ],
        out_specs=pl.BlockSpec((hb, 1), lambda i: (i, 0)),
        out_shape=jax.ShapeDtypeStruct((P_PAD, 1), jnp.float32),
    )(ga, gb, Wh1, bh1.reshape(1, -1), Wh2, bh2.reshape(1, 1))
    return logits.reshape(-1)[:N_PAIRS]


# speedup vs baseline: 11.7422x; 1.0354x over previous
import jax
import jax.numpy as jnp
from jax import lax
from jax.experimental import pallas as pl
from jax.experimental.pallas import tpu as pltpu
from jax.experimental.pallas import tpu_sc as plsc

N_NODES = 10000
N_PAD = 10240
D_FEAT = 128
D1, D2, D3 = 64, 32, 16
N_EDGES = 320000
NW = 32
E_W = 10240
E_PAD = NW * E_W
N_PAIRS = 50000
P_W = 3328
P_PAD = NW * P_W // 2
PW_BLK = 256
ROWS_PER_TILE = N_PAD // 16
ZROWS = 640
CHB = 4


def _mesh():
    return plsc.VectorSubcoreMesh(core_axis_name="c", subcore_axis_name="s")


def _make_agg(d, w, with_deg):
    blocks = E_W // w
    nch = blocks // CHB
    cw = CHB * w
    sb = w // 128
    out_type = [jax.ShapeDtypeStruct((2, N_PAD, d), jnp.float32)]
    scratch = [
        pltpu.VMEM((2 * cw,), jnp.int32),
        pltpu.VMEM((2 * CHB * sb, 128), jnp.int32),
        pltpu.VMEM((2 * w, d), jnp.float32),
        pltpu.VMEM_SHARED((N_PAD, d), jnp.float32),
        pltpu.SemaphoreType.DMA,
        pltpu.SemaphoreType.DMA,
        pltpu.SemaphoreType.DMA,
    ]
    if with_deg:
        out_type.append(jax.ShapeDtypeStruct((2, N_PAD), jnp.float32))
        scratch += [
            pltpu.VMEM((128,), jnp.float32),
            pltpu.VMEM_SHARED((N_PAD,), jnp.float32),
            pltpu.SemaphoreType.DMA,
        ]

    def body(*refs):
        if with_deg:
            (t_hbm, srcm, dstm, zf, zd, part, degp,
             idx_s, idx_d, rows, acc, gsem0, gsem1, stsem,
             ones, dega, dsem) = refs
        else:
            (t_hbm, srcm, dstm, zf, part,
             idx_s, idx_d, rows, acc, gsem0, gsem1, stsem) = refs
        gsem = (gsem0, gsem1)
        c = lax.axis_index("c")
        s = lax.axis_index("s")
        wid = s * 2 + c
        r0 = s * ROWS_PER_TILE
        for k in range(ROWS_PER_TILE // ZROWS):
            pltpu.sync_copy(zf, acc.at[pl.ds(r0 + k * ZROWS, ZROWS)])
            if with_deg:
                pltpu.sync_copy(zd, dega.at[pl.ds(r0 + k * ZROWS, ZROWS)])
        if with_deg:
            for i in range(128 // 16):
                ones[pl.ds(i * 16, 16)] = jnp.full((16,), 1.0, jnp.float32)
        plsc.subcore_barrier()

        def buf(b):
            return rows.at[pl.ds(b * w, w)]

        def _stage_copies(ch, half):
            return (
                pltpu.make_async_copy(
                    srcm.at[pl.ds(wid * E_W + ch * cw, cw)],
                    idx_s.at[pl.ds(half * cw, cw)], stsem),
                pltpu.make_async_copy(
                    dstm.at[pl.ds((wid * E_W + ch * cw) // 128, CHB * sb)],
                    idx_d.at[pl.ds(half * CHB * sb, CHB * sb)], stsem),
            )

        def stage(ch, half):
            for cp in _stage_copies(ch, half):
                cp.start()

        def stage_wait(ch, half):
            for cp in _stage_copies(ch, half):
                cp.wait()

        def gather(pos, b):
            pltpu.async_copy(t_hbm.at[idx_s.at[pl.ds(pos, w)]], buf(b),
                             gsem[b])

        def gather_wait(pos, b):
            pltpu.make_async_copy(t_hbm.at[idx_s.at[pl.ds(pos, w)]], buf(b),
                                  gsem[b]).wait()

        def scatter(half, j, b):
            for q in range(sb):
                drow = half * CHB * sb + j * sb + q
                pltpu.sync_copy(rows.at[pl.ds(b * w + q * 128, 128)],
                                acc.at[idx_d.at[drow]], add=True)
                if with_deg:
                    pltpu.async_copy(ones, dega.at[idx_d.at[drow]], dsem,
                                     add=True)

        stage(0, 0)
        stage_wait(0, 0)
        gather(0, 0)

        def chunk_step(i, carry):
            half = lax.rem(i, 2)
            nxt = lax.rem(i + 1, 2)
            nch_i = lax.rem(i + 1, nch)
            stage(nch_i, nxt)
            for j in range(CHB):
                b = j % 2
                pos = half * cw + j * w
                npos = pos + w if j + 1 < CHB else nxt * cw
                gather_wait(pos, b)
                if j == CHB - 1:
                    stage_wait(nch_i, nxt)
                gather(npos, 1 - b)
                scatter(half, j, b)
            return carry

        lax.fori_loop(0, nch, chunk_step, 0)
        gather_wait(0, 0)
        if with_deg:
            def deg_drain(i, carry):
                pltpu.make_async_copy(ones, dega.at[idx_d.at[0]],
                                      dsem).wait()
                return carry
            lax.fori_loop(0, blocks * sb, deg_drain, 0)
        plsc.subcore_barrier()
        pltpu.sync_copy(acc.at[pl.ds(r0, ROWS_PER_TILE)],
                        part.at[c].at[pl.ds(r0, ROWS_PER_TILE)])
        if with_deg:
            pltpu.sync_copy(dega.at[pl.ds(r0, ROWS_PER_TILE)],
                            degp.at[c].at[pl.ds(r0, ROWS_PER_TILE)])

    def agg(*args):
        return pl.kernel(body, out_type=tuple(out_type), mesh=_mesh(),
                         compiler_params=pltpu.CompilerParams(
                             use_tc_tiling_on_sc=False),
                         scratch_types=tuple(scratch))(*args)
    return agg


def _pair_gather_body(h3, pm, gout, idx, rows, sem0, sem1):
    c = lax.axis_index("c")
    s = lax.axis_index("s")
    wid = s * 2 + c
    pltpu.sync_copy(pm.at[pl.ds(wid * P_W, P_W)], idx)
    base = wid * P_W
    sems = (sem0, sem1)
    nb = P_W // PW_BLK

    def ib(j):
        return idx.at[pl.ds(j * PW_BLK, PW_BLK)]

    def buf(b):
        return rows.at[pl.ds(b * PW_BLK, PW_BLK)]

    def gather(j, b):
        pltpu.async_copy(h3.at[ib(j)], buf(b), sems[b])

    def emit(j, b):
        pltpu.make_async_copy(h3.at[ib(j)], buf(b), sems[b]).wait()
        pltpu.sync_copy(buf(b), gout.at[pl.ds(base + j * PW_BLK, PW_BLK)])

    gather(0, 0)
    gather(1, 1)

    def step(i, carry):
        j = i * 2
        emit(j, 0)
        gather(j + 2, 0)
        emit(j + 1, 1)
        gather(j + 3, 1)
        return carry

    lax.fori_loop(0, (nb - 3) // 2, step, 0)
    emit(nb - 3, 0)
    gather(nb - 1, 0)
    emit(nb - 2, 1)
    emit(nb - 1, 0)


def _pair_gather(h3, pm):
    out_type = jax.ShapeDtypeStruct((NW * P_W, D3), jnp.float32)
    scratch = (
        pltpu.VMEM((P_W,), jnp.int32),
        pltpu.VMEM((2 * PW_BLK, D3), jnp.float32),
        pltpu.SemaphoreType.DMA,
        pltpu.SemaphoreType.DMA,
    )
    return pl.kernel(_pair_gather_body, out_type=out_type, mesh=_mesh(),
                     compiler_params=pltpu.CompilerParams(
                         use_tc_tiling_on_sc=False),
                     scratch_types=scratch)(h3, pm)


def _make_layer_body(act):
    def body(p_ref, degt_ref, w_ref, b_ref, o_ref):
        acc = p_ref[0] + p_ref[1]
        dsum = degt_ref[:, 0:1] + degt_ref[:, 1:2]
        aggn = acc / jnp.maximum(dsum, 1.0)
        out = jnp.dot(aggn, w_ref[...],
                      preferred_element_type=jnp.float32) + b_ref[...]
        o_ref[...] = jnp.maximum(out, 0.0) if act else out
    return body


def _head_body(ga_ref, gb_ref, w1_ref, b1_ref, w2_ref, b2_ref, o_ref):
    feat = jnp.abs(ga_ref[...] - gb_ref[...])
    z = jnp.maximum(
        jnp.dot(feat, w1_ref[...],
                preferred_element_type=jnp.float32) + b1_ref[...], 0.0)
    o_ref[...] = jnp.dot(z, w2_ref[...],
                         preferred_element_type=jnp.float32) + b2_ref[...]


def _tc_call(body, out_shape, *args):
    return pl.pallas_call(
        body, out_shape=jax.ShapeDtypeStruct(out_shape, jnp.float32))(*args)


@jax.jit
def kernel(x, edge_index, pairs, W1, b1, W2, b2, W3, b3, Wh1, bh1, Wh2, bh2):
    x_pad = jnp.pad(x, ((0, N_PAD - N_NODES), (0, 0)))

    n_epad = E_PAD - N_EDGES
    pad_e = (N_NODES + (jnp.arange(n_epad, dtype=jnp.int32) % 128))
    srcm = jnp.concatenate([edge_index[0].astype(jnp.int32), pad_e])
    dstm = jnp.concatenate([edge_index[1].astype(jnp.int32), pad_e]
                           ).reshape(E_PAD // 128, 128)

    n_ppad = P_PAD - N_PAIRS
    pad_p = (N_NODES + (jnp.arange(n_ppad, dtype=jnp.int32) % 128))
    pcat = jnp.concatenate([pairs[:, 0].astype(jnp.int32), pad_p,
                            pairs[:, 1].astype(jnp.int32), pad_p])

    zd = jnp.zeros((ZROWS,), jnp.float32)
    z128 = jnp.zeros((ZROWS, D_FEAT), jnp.float32)
    z64 = jnp.zeros((ZROWS, D1), jnp.float32)
    z32 = jnp.zeros((ZROWS, D2), jnp.float32)

    agg128 = _make_agg(D_FEAT, 128, with_deg=True)
    agg64 = _make_agg(D1, 512, with_deg=False)
    agg32 = _make_agg(D2, 512, with_deg=False)

    p1, degp = agg128(x_pad, srcm, dstm, z128, zd)
    degt = degp.T

    h1 = _tc_call(_make_layer_body(True), (N_PAD, D1), p1, degt, W1,
                  b1.reshape(1, D1))
    p2 = agg64(h1, srcm, dstm, z64)[0]

    h2 = _tc_call(_make_layer_body(True), (N_PAD, D2), p2, degt, W2,
                  b2.reshape(1, D2))
    p3 = agg32(h2, srcm, dstm, z32)[0]

    h3 = _tc_call(_make_layer_body(False), (N_PAD, D3), p3, degt, W3,
                  b3.reshape(1, D3))

    gout = _pair_gather(h3, pcat)

    ga = gout[:P_PAD]
    gb = gout[P_PAD:]

    hb = 4096
    logits = pl.pallas_call(
        _head_body,
        grid=(P_PAD // hb,),
        in_specs=[
            pl.BlockSpec((hb, D3), lambda i: (i, 0)),
            pl.BlockSpec((hb, D3), lambda i: (i, 0)),
            pl.BlockSpec((D3, 32), lambda i: (0, 0)),
            pl.BlockSpec((1, 32), lambda i: (0, 0)),
            pl.BlockSpec((32, 1), lambda i: (0, 0)),
            pl.BlockSpec((1, 1), lambda i: (0, 0)),
        ],
        out_specs=pl.BlockSpec((hb, 1), lambda i: (i, 0)),
        out_shape=jax.ShapeDtypeStruct((P_PAD, 1), jnp.float32),
    )(ga, gb, Wh1, bh1.reshape(1, -1), Wh2, bh2.reshape(1, 1))
    return logits.reshape(-1)[:N_PAIRS]
